# baseline (speedup 1.0000x reference)
import functools

import jax
import jax.numpy as jnp
from jax import lax
from jax.experimental import pallas as pl
from jax.experimental.pallas import tpu as pltpu
from jax.experimental.pallas import tpu_sc as plsc

N_TOKENS = 8192
K_CODES = 8192
D_MODEL = 768

N_SPLIT = 4096
TN = 2048
TK = 2048
LANES = 128
N_TILES = N_SPLIT // TN
K_TILES = K_CODES // TK


HALF_TILES = K_TILES // 2


def _dist_argmin_body(z_ref, cb_ref, zsq_ref, esq_ref,
                      codes_ref, mind_ref, run_min, run_idx,
                      half_v, half_i):
    k = pl.program_id(1)

    @pl.when((k == 0) | (k == HALF_TILES))
    def _():
        run_min[...] = jnp.full((TN, LANES), jnp.inf, jnp.float32)
        run_idx[...] = jnp.zeros((TN, LANES), jnp.int32)

    cross = lax.dot_general(
        z_ref[...], cb_ref[...],
        dimension_numbers=(((1,), (1,)), ((), ())),
        preferred_element_type=jnp.float32)
    d = (zsq_ref[...] + esq_ref[...]) - cross

    def comb(a, b):
        av, ai = a
        bv, bi = b
        take_b = bv < av
        return jnp.where(take_b, bv, av), jnp.where(take_b, bi, ai)

    items = []
    for c in range(TK // LANES):
        dc = d[:, c * LANES:(c + 1) * LANES]
        ic = (lax.broadcasted_iota(jnp.int32, (TN, LANES), 1)
              + (k * TK + c * LANES))
        items.append((dc, ic))
    while len(items) > 1:
        nxt = [comb(items[i], items[i + 1]) for i in range(0, len(items) - 1, 2)]
        if len(items) % 2:
            nxt.append(items[-1])
        items = nxt
    lv, li = items[0]

    rv, ri = comb((run_min[...], run_idx[...]), (lv, li))
    run_min[...] = rv
    run_idx[...] = ri

    def _half_reduce():
        rm = run_min[...]
        rix = run_idx[...]
        gmin = jnp.min(rm, axis=1)
        cand = jnp.where(rm == gmin[:, None], rix, jnp.int32(K_CODES))
        return gmin, jnp.min(cand, axis=1)

    @pl.when(k == HALF_TILES - 1)
    def _():
        v_a, i_a = _half_reduce()
        half_v[...] = v_a
        half_i[...] = i_a

    @pl.when(k == K_TILES - 1)
    def _():
        v_b, i_b = _half_reduce()
        v_a = half_v[...]
        i_a = half_i[...]
        v_a_bf = v_a.astype(jnp.bfloat16).astype(jnp.float32)
        use_b = v_b < v_a_bf
        codes_ref[0, 0, :] = jnp.where(use_b, i_b, i_a)
        d_pick = jnp.where(use_b, v_b, v_a)
        mind_ref[0, 0, :] = d_pick + d_pick


def _codes_and_mindist(z, codebook, z_sq, e_sq):
    return pl.pallas_call(
        _dist_argmin_body,
        grid=(N_TILES, K_TILES),
        in_specs=[
            pl.BlockSpec((TN, D_MODEL), lambda n, k: (n, 0)),
            pl.BlockSpec((TK, D_MODEL), lambda n, k: (k, 0)),
            pl.BlockSpec((TN, 1), lambda n, k: (n, 0)),
            pl.BlockSpec((1, TK), lambda n, k: (0, k)),
        ],
        out_specs=[
            pl.BlockSpec((1, 1, TN), lambda n, k: (n, 0, 0)),
            pl.BlockSpec((1, 1, TN), lambda n, k: (n, 0, 0)),
        ],
        out_shape=[
            jax.ShapeDtypeStruct((N_TILES, 1, TN), jnp.int32),
            jax.ShapeDtypeStruct((N_TILES, 1, TN), jnp.float32),
        ],
        scratch_shapes=[
            pltpu.VMEM((TN, LANES), jnp.float32),
            pltpu.VMEM((TN, LANES), jnp.int32),
            pltpu.VMEM((TN,), jnp.float32),
            pltpu.VMEM((TN,), jnp.int32),
        ],
        compiler_params=pltpu.CompilerParams(
            dimension_semantics=("parallel", "arbitrary")),
    )(z, codebook, z_sq, e_sq)


_SC_WORKERS = 32
_ROWS_PER_WORKER = N_SPLIT // _SC_WORKERS
_GATHER_CHUNK = 64


def _sc_gather(codebook, codes):
    mesh = plsc.VectorSubcoreMesh(core_axis_name="c", subcore_axis_name="s")

    @functools.partial(
        pl.kernel, mesh=mesh,
        out_type=jax.ShapeDtypeStruct((N_SPLIT, D_MODEL), jnp.float32),
        scratch_types=[
            pltpu.VMEM((_GATHER_CHUNK,), jnp.int32),
            pltpu.VMEM((_GATHER_CHUNK, D_MODEL), jnp.float32),
            pltpu.SemaphoreType.DMA,
        ],
    )
    def gather_kernel(table_hbm, idx_hbm, out_hbm, idx_v, rows_v, sem):
        wid = lax.axis_index("s") * 2 + lax.axis_index("c")

        @pl.loop(0, _ROWS_PER_WORKER // _GATHER_CHUNK)
        def _(chunk):
            base = wid * _ROWS_PER_WORKER + chunk * _GATHER_CHUNK
            pltpu.sync_copy(idx_hbm.at[pl.ds(base, _GATHER_CHUNK)], idx_v)
            pltpu.async_copy(table_hbm.at[idx_v], rows_v, sem).wait()
            pltpu.sync_copy(rows_v, out_hbm.at[pl.ds(base, _GATHER_CHUNK)])

    return gather_kernel(codebook, codes)


def kernel(embeddings, codebook):
    B, L, D = embeddings.shape
    z = embeddings.reshape(-1, D)
    z_sq = jnp.sum(z ** 2, axis=-1, keepdims=True)
    e_sq = jnp.sum(codebook ** 2, axis=-1, keepdims=True).T
    zsq_h = 0.5 * z_sq

    codes_parts, mind_parts, quant_parts = [], [], []
    for s in range(N_TOKENS // N_SPLIT):
        lo = s * N_SPLIT
        c3, m3 = _codes_and_mindist(
            z[lo:lo + N_SPLIT], codebook, zsq_h[lo:lo + N_SPLIT], 0.5 * e_sq)
        cflat = c3.reshape(-1)
        codes_parts.append(cflat)
        mind_parts.append(m3)
        quant_parts.append(_sc_gather(codebook, cflat))

    codes = jnp.concatenate(codes_parts)
    quantized = jnp.concatenate(quant_parts, axis=0)

    m = (jnp.sum(mind_parts[0]) + jnp.sum(mind_parts[1])) / (N_TOKENS * D_MODEL)
    vq_loss = m + 0.25 * m

    return (codes.reshape(B, L),
            quantized.reshape(B, L, D),
            vq_loss)


# speedup vs baseline: 1.1936x; 1.1936x over previous
import functools

import jax
import jax.numpy as jnp
from jax import lax
from jax.experimental import pallas as pl
from jax.experimental.pallas import tpu as pltpu
from jax.experimental.pallas import tpu_sc as plsc

N_TOKENS = 8192
K_CODES = 8192
D_MODEL = 768

TN = 2048
TK = 2048
LANES = 128
N_TILES = N_TOKENS // TN
K_TILES = K_CODES // TK


HALF_TILES = K_TILES // 2


def _dist_argmin_body(z_ref, cb_ref, zsq_ref, esq_ref,
                      codes_ref, mind_ref, run_min, run_idx,
                      half_v, half_i):
    k = pl.program_id(1)

    @pl.when((k == 0) | (k == HALF_TILES))
    def _():
        run_min[...] = jnp.full((TN, LANES), jnp.inf, jnp.float32)
        run_idx[...] = jnp.zeros((TN, LANES), jnp.int32)

    cross = lax.dot_general(
        z_ref[...], cb_ref[...],
        dimension_numbers=(((1,), (1,)), ((), ())),
        preferred_element_type=jnp.float32)
    d = (zsq_ref[...] + esq_ref[...]) - cross

    def comb(a, b):
        av, ai = a
        bv, bi = b
        take_b = bv < av
        return jnp.where(take_b, bv, av), jnp.where(take_b, bi, ai)

    items = []
    for c in range(TK // LANES):
        dc = d[:, c * LANES:(c + 1) * LANES]
        ic = (lax.broadcasted_iota(jnp.int32, (TN, LANES), 1)
              + (k * TK + c * LANES))
        items.append((dc, ic))
    while len(items) > 1:
        nxt = [comb(items[i], items[i + 1]) for i in range(0, len(items) - 1, 2)]
        if len(items) % 2:
            nxt.append(items[-1])
        items = nxt
    lv, li = items[0]

    rv, ri = comb((run_min[...], run_idx[...]), (lv, li))
    run_min[...] = rv
    run_idx[...] = ri

    def _half_reduce():
        rm = run_min[...]
        rix = run_idx[...]
        gmin = jnp.min(rm, axis=1)
        cand = jnp.where(rm == gmin[:, None], rix, jnp.int32(K_CODES))
        return gmin, jnp.min(cand, axis=1)

    @pl.when(k == HALF_TILES - 1)
    def _():
        v_a, i_a = _half_reduce()
        half_v[...] = v_a
        half_i[...] = i_a

    @pl.when(k == K_TILES - 1)
    def _():
        v_b, i_b = _half_reduce()
        v_a = half_v[...]
        i_a = half_i[...]
        v_a_bf = v_a.astype(jnp.bfloat16).astype(jnp.float32)
        use_b = v_b < v_a_bf
        codes_ref[0, 0, :] = jnp.where(use_b, i_b, i_a)
        d_pick = jnp.where(use_b, v_b, v_a)
        mind_ref[0, 0, :] = d_pick + d_pick


def _codes_and_mindist(z, codebook, z_sq, e_sq):
    return pl.pallas_call(
        _dist_argmin_body,
        grid=(N_TILES, K_TILES),
        in_specs=[
            pl.BlockSpec((TN, D_MODEL), lambda n, k: (n, 0)),
            pl.BlockSpec((TK, D_MODEL), lambda n, k: (k, 0)),
            pl.BlockSpec((TN, 1), lambda n, k: (n, 0)),
            pl.BlockSpec((1, TK), lambda n, k: (0, k)),
        ],
        out_specs=[
            pl.BlockSpec((1, 1, TN), lambda n, k: (n, 0, 0)),
            pl.BlockSpec((1, 1, TN), lambda n, k: (n, 0, 0)),
        ],
        out_shape=[
            jax.ShapeDtypeStruct((N_TILES, 1, TN), jnp.int32),
            jax.ShapeDtypeStruct((N_TILES, 1, TN), jnp.float32),
        ],
        scratch_shapes=[
            pltpu.VMEM((TN, LANES), jnp.float32),
            pltpu.VMEM((TN, LANES), jnp.int32),
            pltpu.VMEM((TN,), jnp.float32),
            pltpu.VMEM((TN,), jnp.int32),
        ],
        compiler_params=pltpu.CompilerParams(
            dimension_semantics=("parallel", "arbitrary")),
    )(z, codebook, z_sq, e_sq)


_SC_WORKERS = 32
_ROWS_PER_WORKER = N_TOKENS // _SC_WORKERS
_GATHER_CHUNK = 64


def _sc_gather(codebook, codes):
    mesh = plsc.VectorSubcoreMesh(core_axis_name="c", subcore_axis_name="s")

    @functools.partial(
        pl.kernel, mesh=mesh,
        out_type=jax.ShapeDtypeStruct((N_TOKENS, D_MODEL), jnp.float32),
        scratch_types=[
            pltpu.VMEM((_GATHER_CHUNK,), jnp.int32),
            pltpu.VMEM((_GATHER_CHUNK, D_MODEL), jnp.float32),
            pltpu.SemaphoreType.DMA,
        ],
    )
    def gather_kernel(table_hbm, idx_hbm, out_hbm, idx_v, rows_v, sem):
        wid = lax.axis_index("s") * 2 + lax.axis_index("c")

        @pl.loop(0, _ROWS_PER_WORKER // _GATHER_CHUNK)
        def _(chunk):
            base = wid * _ROWS_PER_WORKER + chunk * _GATHER_CHUNK
            pltpu.sync_copy(idx_hbm.at[pl.ds(base, _GATHER_CHUNK)], idx_v)
            pltpu.async_copy(table_hbm.at[idx_v], rows_v, sem).wait()
            pltpu.sync_copy(rows_v, out_hbm.at[pl.ds(base, _GATHER_CHUNK)])

    return gather_kernel(codebook, codes)


def kernel(embeddings, codebook):
    B, L, D = embeddings.shape
    z = embeddings.reshape(-1, D)
    z_sq = jnp.sum(z ** 2, axis=-1, keepdims=True)
    e_sq = jnp.sum(codebook ** 2, axis=-1, keepdims=True).T
    codes3, mind3 = _codes_and_mindist(z, codebook, 0.5 * z_sq, 0.5 * e_sq)
    codes = codes3.reshape(-1)
    quantized = _sc_gather(codebook, codes)

    m = jnp.sum(mind3) / (N_TOKENS * D_MODEL)
    vq_loss = m + 0.25 * m

    return (codes.reshape(B, L),
            quantized.reshape(B, L, D),
            vq_loss)
